# baseline (speedup 1.0000x reference)
import functools

import jax
import jax.numpy as jnp
from jax import lax
from jax.experimental import pallas as pl
from jax.experimental.pallas import tpu as pltpu
from jax.experimental.pallas import tpu_sc as plsc

_L, _D, _B = 160, 512, 1024
_NC, _NS = 2, 16
_NW = _NC * _NS
_RPW = _L // _NW
_BT = 64
_NWR = _B // _BT


def _sc_broadcast(table_hbm, out_hbm, idx_v, tile0, tile1, sem_g, sem_w):
    wid = lax.axis_index("s") * _NC + lax.axis_index("c")
    r0 = wid * _RPW
    tiles = (tile0, tile1)
    pending = []
    for i in range(_RPW):
        r = r0 + i
        tile = tiles[i % 2]
        if i >= 2:
            for cp in pending[(i - 2) * _NWR:(i - 1) * _NWR]:
                cp.wait()
        for j in range(_BT // 16):
            idx_v[pl.ds(j * 16, 16)] = jnp.full((16,), r, jnp.int32)
        pltpu.async_copy(table_hbm.at[idx_v], tile, sem_g).wait()
        for jb in range(_NWR):
            cp = pltpu.make_async_copy(
                tile, out_hbm.at[r, pl.ds(jb * _BT, _BT), :], sem_w)
            cp.start()
            pending.append(cp)
    for cp in pending[(_RPW - 2) * _NWR:]:
        cp.wait()


@functools.partial(jax.jit, static_argnums=())
def _run_sc(embed_weight):
    k = functools.partial(
        pl.kernel,
        mesh=plsc.VectorSubcoreMesh(core_axis_name="c", subcore_axis_name="s"),
        out_type=jax.ShapeDtypeStruct((_L, _B, _D), jnp.float32),
        scratch_types=[
            pltpu.VMEM((_BT,), jnp.int32),
            pltpu.VMEM((_BT, _D), jnp.float32),
            pltpu.VMEM((_BT, _D), jnp.float32),
            pltpu.SemaphoreType.DMA,
            pltpu.SemaphoreType.DMA,
        ],
    )(_sc_broadcast)
    return k(embed_weight)


def kernel(mask, embed_weight):
    del mask
    return _run_sc(embed_weight)


# speedup vs baseline: 1.5044x; 1.5044x over previous
import jax
import jax.numpy as jnp
from jax.experimental import pallas as pl


def _bcast_kernel(w_ref, out_ref):
    out_ref[:] = jnp.broadcast_to(w_ref[:][:, None, :], out_ref.shape)


def kernel(mask, embed_weight):
    l, d = embed_weight.shape
    b = mask.shape[0]
    rows = 8
    bt = 128
    return pl.pallas_call(
        _bcast_kernel,
        grid=(l // rows, b // bt),
        in_specs=[pl.BlockSpec((rows, d), lambda i, j: (i, 0))],
        out_specs=pl.BlockSpec((rows, bt, d), lambda i, j: (i, j, 0)),
        out_shape=jax.ShapeDtypeStruct((l, b, d), embed_weight.dtype),
    )(embed_weight)


# speedup vs baseline: 1.8636x; 1.2388x over previous
import jax
import jax.numpy as jnp
from jax.experimental import pallas as pl


def _bcast_kernel(w_ref, out_ref):
    out_ref[:] = jnp.broadcast_to(w_ref[:][:, None, :], out_ref.shape)


def kernel(mask, embed_weight):
    l, d = embed_weight.shape
    b = mask.shape[0]
    rows = 8
    bt = 512
    return pl.pallas_call(
        _bcast_kernel,
        grid=(l // rows, b // bt),
        in_specs=[pl.BlockSpec((rows, d), lambda i, j: (i, 0))],
        out_specs=pl.BlockSpec((rows, bt, d), lambda i, j: (i, j, 0)),
        out_shape=jax.ShapeDtypeStruct((l, b, d), embed_weight.dtype),
    )(embed_weight)


# speedup vs baseline: 1.8826x; 1.0102x over previous
import jax
import jax.numpy as jnp
from jax.experimental import pallas as pl


def _bcast_kernel(w_ref, out_ref):
    out_ref[:] = jnp.broadcast_to(w_ref[:][:, None, :], out_ref.shape)


def kernel(mask, embed_weight):
    l, d = embed_weight.shape
    b = mask.shape[0]
    rows = 16
    bt = 256
    return pl.pallas_call(
        _bcast_kernel,
        grid=(l // rows, b // bt),
        in_specs=[pl.BlockSpec((rows, d), lambda i, j: (i, 0))],
        out_specs=pl.BlockSpec((rows, bt, d), lambda i, j: (i, j, 0)),
        out_shape=jax.ShapeDtypeStruct((l, b, d), embed_weight.dtype),
    )(embed_weight)
